# baseline (speedup 1.0000x reference)
import functools

import jax
import jax.numpy as jnp
from jax import lax
from jax.experimental import pallas as pl
from jax.experimental.pallas import tpu as pltpu
from jax.experimental.pallas import tpu_sc as plsc

_B = 16384
_D = 32
_PACK = 4
_WIDE = _D * _PACK
_NROWS = 1000000
_QROWS = _NROWS // _PACK
_NC = 2
_NS = 16
_NW = _NC * _NS
_BPW = _B // _NW
_CHUNK = 256
_NCHUNK = _BPW // _CHUNK

_DPB = 2000
_DP_STEPS = _QROWS // _DPB

_HALF = _B // 2
_SCH = 512
_NSCH = _HALF // _SCH


def _depad_body(a_ref, b_ref, c_ref, d_ref, out_ref):
    out_ref[:, 0 * _D:1 * _D] = a_ref[...]
    out_ref[:, 1 * _D:2 * _D] = b_ref[...]
    out_ref[:, 2 * _D:3 * _D] = c_ref[...]
    out_ref[:, 3 * _D:4 * _D] = d_ref[...]


def _tc_depad(emb):
    mk = lambda k: pl.BlockSpec((_DPB, _D), lambda i, k=k: (_DP_STEPS * k + i, 0))
    return pl.pallas_call(
        _depad_body,
        grid=(_DP_STEPS,),
        in_specs=[mk(0), mk(1), mk(2), mk(3)],
        out_specs=pl.BlockSpec((_DPB, _WIDE), lambda i: (i, 0)),
        out_shape=jax.ShapeDtypeStruct((_QROWS, _WIDE), jnp.float32),
    )(emb, emb, emb, emb)


def _scs_gather(emb, idx):
    mesh = plsc.ScalarSubcoreMesh(axis_name="core", num_cores=_NC)

    @functools.partial(
        pl.kernel,
        mesh=mesh,
        out_type=jax.ShapeDtypeStruct((_B, _D), jnp.float32),
        scratch_types=[
            pltpu.SMEM((_SCH,), jnp.int32),
            pltpu.SemaphoreType.DMA,
            pltpu.SemaphoreType.DMA,
        ],
    )
    def k(emb_hbm, idx_hbm, out_hbm, idx_s, sem_r, sem_x):
        cid = lax.axis_index("core")
        base = cid * _HALF

        @pl.loop(0, _NSCH)
        def _(ch):
            cb = base + ch * _SCH
            pltpu.async_copy(idx_hbm.at[pl.ds(cb, _SCH)], idx_s, sem_x).wait()

            @plsc.parallel_loop(0, _SCH, step=8)
            def _(i0):
                for b in range(8):
                    i = i0 + b
                    pltpu.async_copy(emb_hbm.at[idx_s[i]],
                                     out_hbm.at[cb + i], sem_r)

        pltpu.make_async_copy(emb_hbm.at[pl.ds(0, _HALF)],
                              out_hbm.at[pl.ds(base, _HALF)], sem_r).wait()

    return k(emb, idx)


def _sc_stream_gather(emb_w, idx_c):
    mesh = plsc.VectorSubcoreMesh(core_axis_name="c", subcore_axis_name="s")

    @functools.partial(
        pl.kernel,
        mesh=mesh,
        out_type=jax.ShapeDtypeStruct((_B, _WIDE), jnp.float32),
        scratch_types=[
            pltpu.VMEM((_BPW,), jnp.int32),
            pltpu.VMEM((_CHUNK, _WIDE), jnp.float32),
            pltpu.SemaphoreType.DMA,
        ],
    )
    def k(emb_hbm, idx_hbm, out_hbm, idx_v, rows_v, sem):
        wid = lax.axis_index("s") * _NC + lax.axis_index("c")
        base = wid * _BPW
        pltpu.sync_copy(idx_hbm.at[pl.ds(base, _BPW)], idx_v)
        for c in range(_NCHUNK):
            cb = base + c * _CHUNK
            pltpu.async_copy(
                emb_hbm.at[idx_v.at[pl.ds(c * _CHUNK, _CHUNK)]],
                rows_v, sem).wait()
            pltpu.sync_copy(rows_v, out_hbm.at[pl.ds(cb, _CHUNK)])

    return k(emb_w, idx_c)


def _mlp_body(uw_ref, ie_ref, uoh_ref, w1u_ref, w1i_ref, b1_ref,
              w2_ref, b2_ref, w3_ref, b3_ref, wo_ref, bo_ref, out_ref):
    ue = uw_ref[...] * uoh_ref[...]
    x = (jnp.dot(ue, w1u_ref[...], preferred_element_type=jnp.float32)
         + jnp.dot(ie_ref[...], w1i_ref[...], preferred_element_type=jnp.float32)
         + b1_ref[...])
    x = jnp.maximum(x, 0.0)
    x = jnp.dot(x, w2_ref[...], preferred_element_type=jnp.float32) + b2_ref[...]
    x = jnp.maximum(x, 0.0)
    x = jnp.dot(x, w3_ref[...], preferred_element_type=jnp.float32) + b3_ref[...]
    x = jnp.maximum(x, 0.0)
    y = jnp.dot(x, wo_ref[...], preferred_element_type=jnp.float32) + bo_ref[...]
    out_ref[...] = y


_BLK = 2048


def _tc_mlp(uw, ie, uoh, W1, b1, W2, b2, W3, b3, Wout, bout):
    w1u = jnp.tile(W1[:_D], (_PACK, 1))
    w1i = W1[_D:]
    blk = lambda shape: pl.BlockSpec(shape, lambda i: (i, 0))
    full = lambda shape: pl.BlockSpec(shape, lambda i: (0, 0))
    out = pl.pallas_call(
        _mlp_body,
        grid=(_B // _BLK,),
        in_specs=[
            blk((_BLK, _WIDE)), blk((_BLK, _D)), blk((_BLK, _WIDE)),
            full((_WIDE, 32)), full((_D, 32)), full((1, 32)),
            full((32, 16)), full((1, 16)),
            full((16, 8)), full((1, 8)),
            full((8, 1)), full((1, 1)),
        ],
        out_specs=blk((_BLK, 1)),
        out_shape=jax.ShapeDtypeStruct((_B, 1), jnp.float32),
    )(uw, ie, uoh, w1u, w1i, b1[None, :],
      W2, b2[None, :], W3, b3[None, :], Wout, bout[None, :])
    return out[:, 0]


def kernel(user_idx, item_idx, user_emb, item_emb,
           W1, b1, W2, b2, W3, b3, Wout, bout):
    ie = _scs_gather(item_emb, item_idx)
    uemb_w = _tc_depad(user_emb)
    uidx_c = jnp.remainder(user_idx, _QROWS)
    uoh = jnp.repeat(jax.nn.one_hot(user_idx // _QROWS, _PACK,
                                    dtype=jnp.float32), _D, axis=1)
    uw = _sc_stream_gather(uemb_w, uidx_c)
    return _tc_mlp(uw, ie, uoh, W1, b1, W2, b2, W3, b3, Wout, bout)


# speedup vs baseline: 1.1498x; 1.1498x over previous
import functools

import jax
import jax.numpy as jnp
from jax import lax
from jax.experimental import pallas as pl
from jax.experimental.pallas import tpu as pltpu
from jax.experimental.pallas import tpu_sc as plsc

_B = 16384
_D = 32
_PACK = 4
_WIDE = _D * _PACK
_NROWS = 1000000
_NC = 2
_NS = 16
_NW = _NC * _NS
_BPW = _B // _NW
_CHUNK = 256
_NCHUNK = _BPW // _CHUNK

_DP_IN = 8000
_DP_GRID = _NROWS // _DP_IN


_QROWS = _NROWS // _PACK
_DPB = 5000
_DP_STEPS = _QROWS // _DPB


def _depad_body(a_ref, b_ref, c_ref, d_ref, out_ref):
    out_ref[:, 0 * _D:1 * _D] = a_ref[...]
    out_ref[:, 1 * _D:2 * _D] = b_ref[...]
    out_ref[:, 2 * _D:3 * _D] = c_ref[...]
    out_ref[:, 3 * _D:4 * _D] = d_ref[...]


def _tc_depad(emb):
    mk = lambda k: pl.BlockSpec((_DPB, _D), lambda i, k=k: (_DP_STEPS * k + i, 0))
    return pl.pallas_call(
        _depad_body,
        grid=(_DP_STEPS,),
        in_specs=[mk(0), mk(1), mk(2), mk(3)],
        out_specs=pl.BlockSpec((_DPB, _WIDE), lambda i: (i, 0)),
        out_shape=jax.ShapeDtypeStruct((_QROWS, _WIDE), jnp.float32),
    )(emb, emb, emb, emb)


def _sc_gather2(uemb_w, iemb_w, uidx_c, iidx_c):
    mesh = plsc.VectorSubcoreMesh(core_axis_name="c", subcore_axis_name="s")

    @functools.partial(
        pl.kernel,
        mesh=mesh,
        out_type=(
            jax.ShapeDtypeStruct((_B, _WIDE), jnp.float32),
            jax.ShapeDtypeStruct((_B, _WIDE), jnp.float32),
        ),
        scratch_types=[
            pltpu.VMEM((_BPW,), jnp.int32),
            pltpu.VMEM((_BPW,), jnp.int32),
            pltpu.VMEM((_CHUNK, _WIDE), jnp.float32),
            pltpu.VMEM((_CHUNK, _WIDE), jnp.float32),
            pltpu.SemaphoreType.DMA,
            pltpu.SemaphoreType.DMA,
        ],
    )
    def k(uemb_hbm, iemb_hbm, uidx_hbm, iidx_hbm, ue_out, ie_out,
          uidx_v, iidx_v, urows_v, irows_v, sem_u, sem_i):
        wid = lax.axis_index("s") * _NC + lax.axis_index("c")
        base = wid * _BPW
        pltpu.sync_copy(uidx_hbm.at[pl.ds(base, _BPW)], uidx_v)
        pltpu.sync_copy(iidx_hbm.at[pl.ds(base, _BPW)], iidx_v)

        for c in range(_NCHUNK):
            cb = base + c * _CHUNK
            cu = pltpu.async_copy(
                uemb_hbm.at[uidx_v.at[pl.ds(c * _CHUNK, _CHUNK)]],
                urows_v, sem_u)
            ci = pltpu.async_copy(
                iemb_hbm.at[iidx_v.at[pl.ds(c * _CHUNK, _CHUNK)]],
                irows_v, sem_i)
            cu.wait()
            ci.wait()
            pltpu.sync_copy(urows_v, ue_out.at[pl.ds(cb, _CHUNK)])
            pltpu.sync_copy(irows_v, ie_out.at[pl.ds(cb, _CHUNK)])

    return k(uemb_w, iemb_w, uidx_c, iidx_c)


def _mlp_body(uw_ref, iw_ref, uoh_ref, ioh_ref, w1u_ref, w1i_ref, b1_ref,
              w2_ref, b2_ref, w3_ref, b3_ref, wo_ref, bo_ref, out_ref):
    ue = uw_ref[...] * uoh_ref[...]
    ie = iw_ref[...] * ioh_ref[...]
    x = (jnp.dot(ue, w1u_ref[...], preferred_element_type=jnp.float32)
         + jnp.dot(ie, w1i_ref[...], preferred_element_type=jnp.float32)
         + b1_ref[...])
    x = jnp.maximum(x, 0.0)
    x = jnp.dot(x, w2_ref[...], preferred_element_type=jnp.float32) + b2_ref[...]
    x = jnp.maximum(x, 0.0)
    x = jnp.dot(x, w3_ref[...], preferred_element_type=jnp.float32) + b3_ref[...]
    x = jnp.maximum(x, 0.0)
    y = jnp.dot(x, wo_ref[...], preferred_element_type=jnp.float32) + bo_ref[...]
    out_ref[...] = y


_BLK = 2048


def _tc_mlp(uw, iw, uoh, ioh, W1, b1, W2, b2, W3, b3, Wout, bout):
    w1u = jnp.tile(W1[:_D], (_PACK, 1))
    w1i = jnp.tile(W1[_D:], (_PACK, 1))
    blk = lambda shape: pl.BlockSpec(shape, lambda i: (i, 0))
    full = lambda shape: pl.BlockSpec(shape, lambda i: (0, 0))
    out = pl.pallas_call(
        _mlp_body,
        grid=(_B // _BLK,),
        in_specs=[
            blk((_BLK, _WIDE)), blk((_BLK, _WIDE)),
            blk((_BLK, _WIDE)), blk((_BLK, _WIDE)),
            full((_WIDE, 32)), full((_WIDE, 32)), full((1, 32)),
            full((32, 16)), full((1, 16)),
            full((16, 8)), full((1, 8)),
            full((8, 1)), full((1, 1)),
        ],
        out_specs=blk((_BLK, 1)),
        out_shape=jax.ShapeDtypeStruct((_B, 1), jnp.float32),
    )(uw, iw, uoh, ioh, w1u, w1i, b1[None, :],
      W2, b2[None, :], W3, b3[None, :], Wout, bout[None, :])
    return out[:, 0]


def kernel(user_idx, item_idx, user_emb, item_emb,
           W1, b1, W2, b2, W3, b3, Wout, bout):
    uemb_w = jnp.reshape(user_emb, (_QROWS, _WIDE))
    iemb_w = jnp.reshape(item_emb, (_QROWS, _WIDE))
    uidx_c = lax.shift_right_logical(user_idx, 2)
    iidx_c = lax.shift_right_logical(item_idx, 2)
    uoh = jnp.repeat(jax.nn.one_hot(jnp.bitwise_and(user_idx, 3), _PACK,
                                    dtype=jnp.float32), _D, axis=1)
    ioh = jnp.repeat(jax.nn.one_hot(jnp.bitwise_and(item_idx, 3), _PACK,
                                    dtype=jnp.float32), _D, axis=1)
    uw, iw = _sc_gather2(uemb_w, iemb_w, uidx_c, iidx_c)
    return _tc_mlp(uw, iw, uoh, ioh, W1, b1, W2, b2, W3, b3, Wout, bout)


# speedup vs baseline: 1.1508x; 1.0009x over previous
import functools

import jax
import jax.numpy as jnp
from jax import lax
from jax.experimental import pallas as pl
from jax.experimental.pallas import tpu as pltpu
from jax.experimental.pallas import tpu_sc as plsc

_B = 16384
_D = 32
_PACK = 4
_WIDE = _D * _PACK
_NROWS = 1000000
_NC = 2
_NS = 16
_NW = _NC * _NS
_BPW = _B // _NW
_CHUNK = 256
_NCHUNK = _BPW // _CHUNK

_QROWS = _NROWS // _PACK


def _sc_gather2(uemb_w, iemb_w, uidx_c, iidx_c):
    mesh = plsc.VectorSubcoreMesh(core_axis_name="c", subcore_axis_name="s")

    @functools.partial(
        pl.kernel,
        mesh=mesh,
        out_type=(
            jax.ShapeDtypeStruct((_B, _WIDE), jnp.float32),
            jax.ShapeDtypeStruct((_B, _WIDE), jnp.float32),
        ),
        scratch_types=[
            pltpu.VMEM((_BPW,), jnp.int32),
            pltpu.VMEM((_BPW,), jnp.int32),
            pltpu.VMEM((_CHUNK, _WIDE), jnp.float32),
            pltpu.VMEM((_CHUNK, _WIDE), jnp.float32),
            pltpu.SemaphoreType.DMA,
            pltpu.SemaphoreType.DMA,
        ],
    )
    def k(uemb_hbm, iemb_hbm, uidx_hbm, iidx_hbm, ue_out, ie_out,
          uidx_v, iidx_v, urows_v, irows_v, sem_u, sem_i):
        wid = lax.axis_index("s") * _NC + lax.axis_index("c")
        base = wid * _BPW
        pltpu.sync_copy(uidx_hbm.at[pl.ds(base, _BPW)], uidx_v)
        pltpu.sync_copy(iidx_hbm.at[pl.ds(base, _BPW)], iidx_v)

        for c in range(_NCHUNK):
            cb = base + c * _CHUNK
            cu = pltpu.async_copy(
                uemb_hbm.at[uidx_v.at[pl.ds(c * _CHUNK, _CHUNK)]],
                urows_v, sem_u)
            ci = pltpu.async_copy(
                iemb_hbm.at[iidx_v.at[pl.ds(c * _CHUNK, _CHUNK)]],
                irows_v, sem_i)
            cu.wait()
            ci.wait()
            pltpu.sync_copy(urows_v, ue_out.at[pl.ds(cb, _CHUNK)])
            pltpu.sync_copy(irows_v, ie_out.at[pl.ds(cb, _CHUNK)])

    return k(uemb_w, iemb_w, uidx_c, iidx_c)


def _mlp_body(uw_ref, iw_ref, uoh_ref, ioh_ref, w1u_ref, w1i_ref, b1_ref,
              w2_ref, b2_ref, w3_ref, b3_ref, wo_ref, bo_ref, out_ref):
    ue = uw_ref[...] * uoh_ref[...]
    ie = iw_ref[...] * ioh_ref[...]
    x = (jnp.dot(ue, w1u_ref[...], preferred_element_type=jnp.float32)
         + jnp.dot(ie, w1i_ref[...], preferred_element_type=jnp.float32)
         + b1_ref[...])
    x = jnp.maximum(x, 0.0)
    x = jnp.dot(x, w2_ref[...], preferred_element_type=jnp.float32) + b2_ref[...]
    x = jnp.maximum(x, 0.0)
    x = jnp.dot(x, w3_ref[...], preferred_element_type=jnp.float32) + b3_ref[...]
    x = jnp.maximum(x, 0.0)
    y = jnp.dot(x, wo_ref[...], preferred_element_type=jnp.float32) + bo_ref[...]
    out_ref[...] = y


_BLK = 2048


def _tc_mlp(uw, iw, uoh, ioh, W1, b1, W2, b2, W3, b3, Wout, bout):
    w1u = jnp.tile(W1[:_D], (_PACK, 1))
    w1i = jnp.tile(W1[_D:], (_PACK, 1))
    blk = lambda shape: pl.BlockSpec(shape, lambda i: (i, 0))
    full = lambda shape: pl.BlockSpec(shape, lambda i: (0, 0))
    out = pl.pallas_call(
        _mlp_body,
        grid=(_B // _BLK,),
        in_specs=[
            blk((_BLK, _WIDE)), blk((_BLK, _WIDE)),
            blk((_BLK, _WIDE)), blk((_BLK, _WIDE)),
            full((_WIDE, 32)), full((_WIDE, 32)), full((1, 32)),
            full((32, 16)), full((1, 16)),
            full((16, 8)), full((1, 8)),
            full((8, 1)), full((1, 1)),
        ],
        out_specs=blk((_BLK, 1)),
        out_shape=jax.ShapeDtypeStruct((_B, 1), jnp.float32),
    )(uw, iw, uoh, ioh, w1u, w1i, b1[None, :],
      W2, b2[None, :], W3, b3[None, :], Wout, bout[None, :])
    return out[:, 0]


def kernel(user_idx, item_idx, user_emb, item_emb,
           W1, b1, W2, b2, W3, b3, Wout, bout):
    uemb_w = jnp.reshape(user_emb, (_QROWS, _WIDE))
    iemb_w = jnp.reshape(item_emb, (_QROWS, _WIDE))
    uidx_c = lax.shift_right_logical(user_idx, 2)
    iidx_c = lax.shift_right_logical(item_idx, 2)
    uoh = jnp.repeat(jax.nn.one_hot(jnp.bitwise_and(user_idx, 3), _PACK,
                                    dtype=jnp.float32), _D, axis=1)
    ioh = jnp.repeat(jax.nn.one_hot(jnp.bitwise_and(item_idx, 3), _PACK,
                                    dtype=jnp.float32), _D, axis=1)
    uw, iw = _sc_gather2(uemb_w, iemb_w, uidx_c, iidx_c)
    return _tc_mlp(uw, iw, uoh, ioh, W1, b1, W2, b2, W3, b3, Wout, bout)
